# Initial kernel scaffold; baseline (speedup 1.0000x reference)
import functools

import jax
import jax.numpy as jnp
from jax import lax
from jax.experimental import pallas as pl
from jax.experimental.pallas import tpu as pltpu
from jax.experimental.pallas import tpu_sc as plsc

N = 10000
D = 256
DH = 128
E = 160000
NUM_LAYERS = 5
NC = 2
NS = 16

EPS = E // NS
CHUNK = 80
NCHUNK = EPS // CHUNK
RPS = N // NS

EPC = E // NC
DPS = EPC // NS
DCHUNK = 40
DNCHUNK = DPS // DCHUNK
CW = 16

BN = 1000

_MESH = plsc.VectorSubcoreMesh(core_axis_name="c", subcore_axis_name="s")


def _deg_counts(dst):
    ones = jnp.ones((DCHUNK, CW), jnp.float32)
    zeros = jnp.zeros((RPS, CW), jnp.float32)

    @functools.partial(
        pl.kernel,
        mesh=_MESH,
        out_type=[jax.ShapeDtypeStruct((N, CW), jnp.float32)] * NC,
        scratch_types=[
            pltpu.VMEM((DCHUNK,), jnp.int32),
            pltpu.VMEM((DCHUNK, CW), jnp.float32),
            pltpu.VMEM_SHARED((N, CW), jnp.float32),
            pltpu.SemaphoreType.DMA,
        ],
    )
    def k(dst_hbm, ones_hbm, zeros_hbm, cnt0_hbm, cnt1_hbm,
          idx_v, ones_v, acc_sh, sem):
        c = lax.axis_index("c")
        s = lax.axis_index("s")
        row = pl.ds(s * RPS, RPS)
        pltpu.sync_copy(zeros_hbm, acc_sh.at[row])
        pltpu.sync_copy(ones_hbm, ones_v)
        plsc.subcore_barrier()
        base = c * EPC + s * DPS

        @pl.loop(0, DNCHUNK)
        def _(g):
            pltpu.sync_copy(dst_hbm.at[pl.ds(base + g * DCHUNK, DCHUNK)],
                            idx_v)
            pltpu.sync_copy(ones_v, acc_sh.at[idx_v], add=True)

        plsc.subcore_barrier()

        @pl.when(c == 0)
        def _():
            pltpu.sync_copy(acc_sh.at[row], cnt0_hbm.at[row])

        @pl.when(c == 1)
        def _():
            pltpu.sync_copy(acc_sh.at[row], cnt1_hbm.at[row])

    return k(dst, ones, zeros)


def _aggregate(h2a, h2b, src, dst):
    zeros = jnp.zeros((RPS, DH), jnp.float32)

    @functools.partial(
        pl.kernel,
        mesh=_MESH,
        out_type=[jax.ShapeDtypeStruct((N, DH), jnp.float32)] * NC,
        scratch_types=[
            pltpu.VMEM((CHUNK,), jnp.int32),
            pltpu.VMEM((CHUNK,), jnp.int32),
            pltpu.VMEM((CHUNK, DH), jnp.float32),
            pltpu.VMEM_SHARED((N, DH), jnp.float32),
            pltpu.SemaphoreType.DMA,
        ],
    )
    def k(h2a_hbm, h2b_hbm, src_hbm, dst_hbm, zeros_hbm,
          outa_hbm, outb_hbm, src_v, dst_v, rows_v, acc_sh, sem):
        c = lax.axis_index("c")
        s = lax.axis_index("s")
        row = pl.ds(s * RPS, RPS)
        pltpu.sync_copy(zeros_hbm, acc_sh.at[row])
        plsc.subcore_barrier()

        def run(table_hbm):
            base = s * EPS

            @pl.loop(0, NCHUNK)
            def _(g):
                off = base + g * CHUNK
                pltpu.sync_copy(src_hbm.at[pl.ds(off, CHUNK)], src_v)
                pltpu.sync_copy(dst_hbm.at[pl.ds(off, CHUNK)], dst_v)
                pltpu.async_copy(table_hbm.at[src_v], rows_v, sem).wait()
                pltpu.sync_copy(rows_v, acc_sh.at[dst_v], add=True)

        @pl.when(c == 0)
        def _():
            run(h2a_hbm)

        @pl.when(c == 1)
        def _():
            run(h2b_hbm)

        plsc.subcore_barrier()

        @pl.when(c == 0)
        def _():
            pltpu.sync_copy(acc_sh.at[row], outa_hbm.at[row])

        @pl.when(c == 1)
        def _():
            pltpu.sync_copy(acc_sh.at[row], outb_hbm.at[row])

    return k(h2a, h2b, src, dst, zeros)


def _dinv_from_counts(cnt0, cnt1):
    def body(c0, c1, o):
        deg = c0[:, 0:1] + c1[:, 0:1] + 1.0
        o[...] = lax.rsqrt(deg)

    return pl.pallas_call(
        body,
        grid=(N // BN,),
        in_specs=[pl.BlockSpec((BN, CW), lambda i: (i, 0))] * 2,
        out_specs=pl.BlockSpec((BN, 1), lambda i: (i, 0)),
        out_shape=jax.ShapeDtypeStruct((N, 1), jnp.float32),
    )(cnt0, cnt1)


def _matmul_first(x, W, dinv):
    def body(x_r, w_r, dv_r, oa, ob):
        h = lax.dot_general(x_r[...], w_r[...], (((1,), (0,)), ((), ())),
                            precision=lax.Precision.HIGHEST,
                            preferred_element_type=jnp.float32)
        h = h * dv_r[...]
        oa[...] = h[:, :DH]
        ob[...] = h[:, DH:]

    return pl.pallas_call(
        body,
        grid=(N // BN,),
        in_specs=[pl.BlockSpec((BN, D), lambda i: (i, 0)),
                  pl.BlockSpec((D, D), lambda i: (0, 0)),
                  pl.BlockSpec((BN, 1), lambda i: (i, 0))],
        out_specs=[pl.BlockSpec((BN, DH), lambda i: (i, 0))] * 2,
        out_shape=[jax.ShapeDtypeStruct((N, DH), jnp.float32)] * 2,
    )(x, W, dinv)


def _matmul_mid(acca, accb, h2a, h2b, dinv, b_prev, W):
    def body(aa, ab, ha, hb, dv_r, b_r, w_r, oa, ob):
        dv = dv_r[...]
        za = dv * (aa[...] + ha[...]) + b_r[:, :DH]
        zb = dv * (ab[...] + hb[...]) + b_r[:, DH:]
        z = jnp.maximum(jnp.concatenate([za, zb], axis=1), 0.0)
        h = lax.dot_general(z, w_r[...], (((1,), (0,)), ((), ())),
                            precision=lax.Precision.HIGHEST,
                            preferred_element_type=jnp.float32)
        h = h * dv
        oa[...] = h[:, :DH]
        ob[...] = h[:, DH:]

    half = pl.BlockSpec((BN, DH), lambda i: (i, 0))
    return pl.pallas_call(
        body,
        grid=(N // BN,),
        in_specs=[half, half, half, half,
                  pl.BlockSpec((BN, 1), lambda i: (i, 0)),
                  pl.BlockSpec((1, D), lambda i: (0, 0)),
                  pl.BlockSpec((D, D), lambda i: (0, 0))],
        out_specs=[half, half],
        out_shape=[jax.ShapeDtypeStruct((N, DH), jnp.float32)] * 2,
    )(acca, accb, h2a, h2b, dinv, b_prev, W)


def _epilogue_last(acca, accb, h2a, h2b, dinv, b):
    def body(aa, ab, ha, hb, dv_r, b_r, o):
        dv = dv_r[...]
        oa = dv * (aa[...] + ha[...]) + b_r[:, :DH]
        ob = dv * (ab[...] + hb[...]) + b_r[:, DH:]
        o[...] = jnp.concatenate([oa, ob], axis=1)

    half = pl.BlockSpec((BN, DH), lambda i: (i, 0))
    return pl.pallas_call(
        body,
        grid=(N // BN,),
        in_specs=[half, half, half, half,
                  pl.BlockSpec((BN, 1), lambda i: (i, 0)),
                  pl.BlockSpec((1, D), lambda i: (0, 0))],
        out_specs=pl.BlockSpec((BN, D), lambda i: (i, 0)),
        out_shape=jax.ShapeDtypeStruct((N, D), jnp.float32),
    )(acca, accb, h2a, h2b, dinv, b)


def kernel(x, edge_index, W0, b0, W1, b1, W2, b2, W3, b3, W4, b4):
    src = edge_index[0]
    dst = edge_index[1]
    Ws = [W0, W1, W2, W3, W4]
    bs = [b.reshape(1, D) for b in (b0, b1, b2, b3, b4)]

    cnt0, cnt1 = _deg_counts(dst)
    dinv = _dinv_from_counts(cnt0, cnt1)

    h2a, h2b = _matmul_first(x, Ws[0], dinv)
    for i in range(1, NUM_LAYERS):
        acca, accb = _aggregate(h2a, h2b, src, dst)
        h2a, h2b = _matmul_mid(acca, accb, h2a, h2b, dinv, bs[i - 1], Ws[i])
    acca, accb = _aggregate(h2a, h2b, src, dst)
    return _epilogue_last(acca, accb, h2a, h2b, dinv, bs[-1])


# speedup vs baseline: 6.2697x; 6.2697x over previous
import functools

import jax
import jax.numpy as jnp
from jax import lax
from jax.experimental import pallas as pl
from jax.experimental.pallas import tpu as pltpu
from jax.experimental.pallas import tpu_sc as plsc

N = 10000
D = 256
DH = 128
E = 160000
NUM_LAYERS = 5
NC = 2
NS = 16

EPS = E // NS
CHUNK = 80
NCHUNK = EPS // CHUNK
NP = 10240
RPS = NP // NS

EPC = E // NC
DPS = EPC // NS
DCHUNK = 40
DNCHUNK = DPS // DCHUNK
CW = 128

BN = 1000

_MESH = plsc.VectorSubcoreMesh(core_axis_name="c", subcore_axis_name="s")


def _deg_counts(dst):
    ones = jnp.ones((DCHUNK, CW), jnp.float32)
    zeros = jnp.zeros((RPS, CW), jnp.float32)

    @functools.partial(
        pl.kernel,
        mesh=_MESH,
        out_type=jax.ShapeDtypeStruct((NC, NP, CW), jnp.float32),
        scratch_types=[
            pltpu.VMEM((DCHUNK,), jnp.int32),
            pltpu.VMEM((DCHUNK, CW), jnp.float32),
            pltpu.VMEM_SHARED((NP, CW), jnp.float32),
            pltpu.SemaphoreType.DMA,
        ],
    )
    def k(dst_hbm, ones_hbm, zeros_hbm, cnt_hbm, idx_v, ones_v, acc_sh, sem):
        c = lax.axis_index("c")
        s = lax.axis_index("s")
        row = pl.ds(s * RPS, RPS)
        pltpu.sync_copy(zeros_hbm, acc_sh.at[row])
        pltpu.sync_copy(ones_hbm, ones_v)
        plsc.subcore_barrier()
        base = c * EPC + s * DPS

        @pl.loop(0, DNCHUNK)
        def _(g):
            pltpu.sync_copy(dst_hbm.at[pl.ds(base + g * DCHUNK, DCHUNK)],
                            idx_v)
            pltpu.sync_copy(ones_v, acc_sh.at[idx_v], add=True)

        plsc.subcore_barrier()
        pltpu.sync_copy(acc_sh.at[row], cnt_hbm.at[c, row])

    return k(dst, ones, zeros)


def _aggregate(h2s, sidx, dst):
    zeros = jnp.zeros((RPS, DH), jnp.float32)

    @functools.partial(
        pl.kernel,
        mesh=_MESH,
        out_type=jax.ShapeDtypeStruct((NC, NP, DH), jnp.float32),
        scratch_types=[
            pltpu.VMEM((CHUNK,), jnp.int32),
            pltpu.VMEM((CHUNK,), jnp.int32),
            pltpu.VMEM((CHUNK, DH), jnp.float32),
            pltpu.VMEM_SHARED((NP, DH), jnp.float32),
            pltpu.SemaphoreType.DMA,
        ],
    )
    def k(h2s_hbm, sidx_hbm, dst_hbm, zeros_hbm, out_hbm,
          src_v, dst_v, rows_v, acc_sh, sem):
        c = lax.axis_index("c")
        s = lax.axis_index("s")
        row = pl.ds(s * RPS, RPS)
        pltpu.sync_copy(zeros_hbm, acc_sh.at[row])
        plsc.subcore_barrier()
        base = c * E + s * EPS

        @pl.loop(0, NCHUNK)
        def _(g):
            off = base + g * CHUNK
            pltpu.sync_copy(sidx_hbm.at[pl.ds(off, CHUNK)], src_v)
            pltpu.sync_copy(dst_hbm.at[pl.ds(s * EPS + g * CHUNK, CHUNK)],
                            dst_v)
            pltpu.async_copy(h2s_hbm.at[src_v], rows_v, sem).wait()
            pltpu.sync_copy(rows_v, acc_sh.at[dst_v], add=True)

        plsc.subcore_barrier()
        pltpu.sync_copy(acc_sh.at[row], out_hbm.at[c, row])

    return k(h2s, sidx, dst, zeros)


def _dinv_from_counts(cnt):
    def body(c_r, o):
        deg = c_r[0, :, 0:1] + c_r[1, :, 0:1] + 1.0
        o[...] = lax.rsqrt(deg)

    return pl.pallas_call(
        body,
        grid=(N // BN,),
        in_specs=[pl.BlockSpec((NC, BN, CW), lambda i: (0, i, 0))],
        out_specs=pl.BlockSpec((BN, 1), lambda i: (i, 0)),
        out_shape=jax.ShapeDtypeStruct((N, 1), jnp.float32),
    )(cnt)


def _matmul_first(x, W, dinv):
    def body(x_r, w_r, dv_r, o):
        h = lax.dot_general(x_r[...], w_r[...], (((1,), (0,)), ((), ())),
                            precision=lax.Precision.HIGHEST,
                            preferred_element_type=jnp.float32)
        h = h * dv_r[...]
        o[0] = h[:, :DH]
        o[1] = h[:, DH:]

    return pl.pallas_call(
        body,
        grid=(N // BN,),
        in_specs=[pl.BlockSpec((BN, D), lambda i: (i, 0)),
                  pl.BlockSpec((D, D), lambda i: (0, 0)),
                  pl.BlockSpec((BN, 1), lambda i: (i, 0))],
        out_specs=pl.BlockSpec((NC, BN, DH), lambda i: (0, i, 0)),
        out_shape=jax.ShapeDtypeStruct((NC, N, DH), jnp.float32),
    )(x, W, dinv)


def _matmul_mid(acc, h2, dinv, b_prev, W):
    def body(a_r, h_r, dv_r, b_r, w_r, o):
        dv = dv_r[...]
        za = dv * (a_r[0] + h_r[0]) + b_r[:, :DH]
        zb = dv * (a_r[1] + h_r[1]) + b_r[:, DH:]
        z = jnp.maximum(jnp.concatenate([za, zb], axis=1), 0.0)
        h = lax.dot_general(z, w_r[...], (((1,), (0,)), ((), ())),
                            precision=lax.Precision.HIGHEST,
                            preferred_element_type=jnp.float32)
        h = h * dv
        o[0] = h[:, :DH]
        o[1] = h[:, DH:]

    return pl.pallas_call(
        body,
        grid=(N // BN,),
        in_specs=[pl.BlockSpec((NC, BN, DH), lambda i: (0, i, 0)),
                  pl.BlockSpec((NC, BN, DH), lambda i: (0, i, 0)),
                  pl.BlockSpec((BN, 1), lambda i: (i, 0)),
                  pl.BlockSpec((1, D), lambda i: (0, 0)),
                  pl.BlockSpec((D, D), lambda i: (0, 0))],
        out_specs=pl.BlockSpec((NC, BN, DH), lambda i: (0, i, 0)),
        out_shape=jax.ShapeDtypeStruct((NC, N, DH), jnp.float32),
    )(acc, h2, dinv, b_prev, W)


def _epilogue_last(acc, h2, dinv, b):
    def body(a_r, h_r, dv_r, b_r, o):
        dv = dv_r[...]
        oa = dv * (a_r[0] + h_r[0]) + b_r[:, :DH]
        ob = dv * (a_r[1] + h_r[1]) + b_r[:, DH:]
        o[...] = jnp.concatenate([oa, ob], axis=1)

    return pl.pallas_call(
        body,
        grid=(N // BN,),
        in_specs=[pl.BlockSpec((NC, BN, DH), lambda i: (0, i, 0)),
                  pl.BlockSpec((NC, BN, DH), lambda i: (0, i, 0)),
                  pl.BlockSpec((BN, 1), lambda i: (i, 0)),
                  pl.BlockSpec((1, D), lambda i: (0, 0))],
        out_specs=pl.BlockSpec((BN, D), lambda i: (i, 0)),
        out_shape=jax.ShapeDtypeStruct((N, D), jnp.float32),
    )(acc, h2, dinv, b)


def kernel(x, edge_index, W0, b0, W1, b1, W2, b2, W3, b3, W4, b4):
    src = edge_index[0]
    dst = edge_index[1]
    sidx = jnp.concatenate([src, src + N])
    Ws = [W0, W1, W2, W3, W4]
    bs = [b.reshape(1, D) for b in (b0, b1, b2, b3, b4)]

    cnt = _deg_counts(dst)
    dinv = _dinv_from_counts(cnt)

    h2 = _matmul_first(x, Ws[0], dinv)
    for i in range(1, NUM_LAYERS):
        acc = _aggregate(h2.reshape(NC * N, DH), sidx, dst)
        h2 = _matmul_mid(acc, h2, dinv, bs[i - 1], Ws[i])
    acc = _aggregate(h2.reshape(NC * N, DH), sidx, dst)
    return _epilogue_last(acc, h2, dinv, bs[-1])
